# Initial kernel scaffold; baseline (speedup 1.0000x reference)
import jax
import jax.numpy as jnp
from jax.experimental import pallas as pl


def kernel(xb, conv1_w, conv1_scale, conv1_shift, conv2_w, conv2_scale, conv2_shift, res1a_w, res1a_scale, res1a_shift, res1b_w, res1b_scale, res1b_shift, conv3_w, conv3_scale, conv3_shift, conv4_w, conv4_scale, conv4_shift, res2a_w, res2a_scale, res2a_shift, res2b_w, res2b_scale, res2b_shift, fc1_w, fc1_b, fc2_w, fc2_b):
    raise NotImplementedError("write your pallas kernel here")


# speedup vs baseline: 1.4461x; 1.4461x over previous
import jax
import jax.numpy as jnp
from jax.experimental import pallas as pl
from jax.experimental.pallas import tpu as pltpu

_SLOPE = 0.01
_VMEM = 48 * 1024 * 1024


def _lrelu(y):
    return jnp.where(y >= 0, y, _SLOPE * y)


def _make_plain_body(Bt, H, W, C, n_extra):
    M = Bt * H * W

    def body(plane_ref, w_ref, scale_ref, shift_ref, *rest):
        if n_extra:
            res_ref, o_ref = rest
        else:
            (o_ref,) = rest
        taps = []
        for dy in range(3):
            for dx in range(3):
                s = plane_ref[:, dy:dy + H, dx:dx + W, :]
                taps.append(s.reshape(M, C))
        patch = jnp.concatenate(taps, axis=-1)
        y = jnp.dot(patch, w_ref[...], preferred_element_type=jnp.float32)
        y = _lrelu(y * scale_ref[...] + shift_ref[...])
        if n_extra:
            y = y + res_ref[...].astype(jnp.float32)
        o_ref[...] = y.astype(o_ref.dtype)

    return body


def _make_pool_body(Bt, Ho, Wo, C):
    M = Bt * Ho * Wo

    def body(p00, p01, p10, p11, w_ref, scale_ref, shift_ref, o_ref):
        planes = ((p00, p01), (p10, p11))
        w = w_ref[...]
        scale = scale_ref[...]
        shift = shift_ref[...]
        acc = None
        for iy in range(2):
            for ix in range(2):
                taps = []
                for dy in range(3):
                    for dx in range(3):
                        tr = iy + dy - 1
                        tc = ix + dx - 1
                        pr, ur = tr & 1, tr >> 1
                        pc, uc = tc & 1, tc >> 1
                        ref = planes[pr][pc]
                        s = ref[:, 1 + ur:1 + ur + Ho, 1 + uc:1 + uc + Wo, :]
                        taps.append(s.reshape(M, C))
                patch = jnp.concatenate(taps, axis=-1)
                y = jnp.dot(patch, w, preferred_element_type=jnp.float32)
                y = _lrelu(y * scale + shift)
                acc = y if acc is None else jnp.maximum(acc, y)
        o_ref[...] = acc.astype(o_ref.dtype)

    return body


def _conv1_body(x_ref, w_ref, scale_ref, shift_ref, o_ref):
    y = jnp.dot(x_ref[...], w_ref[...], preferred_element_type=jnp.float32)
    y = _lrelu(y * scale_ref[...] + shift_ref[...])
    z = jnp.maximum(y[:, :128], y[:, 128:])
    z = jnp.maximum(z[:, :64], z[:, 64:])
    o_ref[...] = z.astype(o_ref.dtype)


def _fc_body(x_ref, w1_ref, b1_ref, w2_ref, b2_ref, o_ref):
    x = x_ref[...].astype(jnp.float32)
    h = jnp.dot(x, w1_ref[...], preferred_element_type=jnp.float32) + b1_ref[...]
    logits = jnp.dot(h, w2_ref[...], preferred_element_type=jnp.float32) + b2_ref[...]
    m = jnp.max(logits, axis=-1, keepdims=True)
    e = jnp.exp(logits - m)
    o_ref[...] = e / jnp.sum(e, axis=-1, keepdims=True)


def _conv_plain(x, w_flat, scale, shift, B, H, W, C, Cout, Bt, residual=None):
    xp = jnp.pad(x.reshape(B, H, W, C), ((0, 0), (1, 1), (1, 1), (0, 0)))
    M = B * H * W
    Mt = Bt * H * W
    n_extra = 0 if residual is None else 1
    inputs = [xp, w_flat[:9 * C], scale, shift]
    in_specs = [
        pl.BlockSpec((Bt, H + 2, W + 2, C), lambda i: (i, 0, 0, 0)),
        pl.BlockSpec((9 * C, Cout), lambda i: (0, 0)),
        pl.BlockSpec((1, Cout), lambda i: (0, 0)),
        pl.BlockSpec((1, Cout), lambda i: (0, 0)),
    ]
    if residual is not None:
        inputs.append(residual)
        in_specs.append(pl.BlockSpec((Mt, Cout), lambda i: (i, 0)))
    return pl.pallas_call(
        _make_plain_body(Bt, H, W, C, n_extra),
        out_shape=jax.ShapeDtypeStruct((M, Cout), jnp.bfloat16),
        grid=(B // Bt,),
        in_specs=in_specs,
        out_specs=pl.BlockSpec((Mt, Cout), lambda i: (i, 0)),
        compiler_params=pltpu.CompilerParams(
            dimension_semantics=("parallel",), vmem_limit_bytes=_VMEM),
    )(*inputs)


def _conv_pool(x, w_flat, scale, shift, B, H, W, C, Cout, Bt):
    Ho, Wo = H // 2, W // 2
    x4 = x.reshape(B, H, W, C)
    planes = [
        jnp.pad(x4[:, p::2, q::2, :], ((0, 0), (1, 1), (1, 1), (0, 0)))
        for p in range(2) for q in range(2)
    ]
    M = B * Ho * Wo
    Mt = Bt * Ho * Wo
    pspec = pl.BlockSpec((Bt, Ho + 2, Wo + 2, C), lambda i: (i, 0, 0, 0))
    return pl.pallas_call(
        _make_pool_body(Bt, Ho, Wo, C),
        out_shape=jax.ShapeDtypeStruct((M, Cout), jnp.bfloat16),
        grid=(B // Bt,),
        in_specs=[pspec, pspec, pspec, pspec,
                  pl.BlockSpec((9 * C, Cout), lambda i: (0, 0)),
                  pl.BlockSpec((1, Cout), lambda i: (0, 0)),
                  pl.BlockSpec((1, Cout), lambda i: (0, 0))],
        out_specs=pl.BlockSpec((Mt, Cout), lambda i: (i, 0)),
        compiler_params=pltpu.CompilerParams(
            dimension_semantics=("parallel",), vmem_limit_bytes=_VMEM),
    )(*planes, w_flat[:9 * C], scale, shift)


def kernel(xb, conv1_w, conv1_scale, conv1_shift, conv2_w, conv2_scale,
           conv2_shift, res1a_w, res1a_scale, res1a_shift, res1b_w,
           res1b_scale, res1b_shift, conv3_w, conv3_scale, conv3_shift,
           conv4_w, conv4_scale, conv4_shift, res2a_w, res2a_scale,
           res2a_shift, res2b_w, res2b_scale, res2b_shift,
           fc1_w, fc1_b, fc2_w, fc2_b):
    B = xb.shape[0]

    x = xb.reshape(B, 48, 48).astype(jnp.bfloat16)
    xp = jnp.pad(x, ((0, 0), (1, 1), (1, 1)))
    cols = jnp.stack(
        [xp[:, r:r + 48:2, s:s + 48:2] for r in range(4) for s in range(4)],
        axis=-1)
    cols = cols.reshape(B * 576, 16)
    w9 = conv1_w[:9].astype(jnp.float32).reshape(3, 3, 64)
    w16 = jnp.zeros((4, 4, 4, 64), jnp.float32)
    for iy in range(2):
        for ix in range(2):
            g = iy * 2 + ix
            for dy in range(3):
                for dx in range(3):
                    w16 = w16.at[iy + dy, ix + dx, g].set(w9[dy, dx])
    w16 = w16.reshape(16, 256).astype(jnp.bfloat16)
    sc4 = jnp.tile(conv1_scale, (1, 4))
    sh4 = jnp.tile(conv1_shift, (1, 4))
    TM = min(4608, B * 576)
    out = pl.pallas_call(
        _conv1_body,
        out_shape=jax.ShapeDtypeStruct((B * 576, 64), jnp.bfloat16),
        grid=(B * 576 // TM,),
        in_specs=[pl.BlockSpec((TM, 16), lambda i: (i, 0)),
                  pl.BlockSpec((16, 256), lambda i: (0, 0)),
                  pl.BlockSpec((1, 256), lambda i: (0, 0)),
                  pl.BlockSpec((1, 256), lambda i: (0, 0))],
        out_specs=pl.BlockSpec((TM, 64), lambda i: (i, 0)),
        compiler_params=pltpu.CompilerParams(
            dimension_semantics=("parallel",), vmem_limit_bytes=_VMEM),
    )(cols, w16, sc4, sh4)

    out = _conv_pool(out, conv2_w, conv2_scale, conv2_shift,
                     B, 24, 24, 64, 128, Bt=min(32, B))
    r = _conv_plain(out, res1a_w, res1a_scale, res1a_shift,
                    B, 12, 12, 128, 128, Bt=min(16, B))
    out = _conv_plain(r, res1b_w, res1b_scale, res1b_shift,
                      B, 12, 12, 128, 128, Bt=min(16, B), residual=out)
    out = _conv_pool(out, conv3_w, conv3_scale, conv3_shift,
                     B, 12, 12, 128, 256, Bt=min(32, B))
    out = _conv_pool(out, conv4_w, conv4_scale, conv4_shift,
                     B, 6, 6, 256, 512, Bt=min(64, B))
    r = _conv_plain(out, res2a_w, res2a_scale, res2a_shift,
                    B, 3, 3, 512, 512, Bt=min(64, B))
    out = _conv_plain(r, res2b_w, res2b_scale, res2b_shift,
                      B, 3, 3, 512, 512, Bt=min(64, B), residual=out)

    feats = out.reshape(B, 9 * 512)
    w1r = jnp.tile(fc1_w, (9, 1)) / 9.0
    TB = min(256, B)
    probs = pl.pallas_call(
        _fc_body,
        out_shape=jax.ShapeDtypeStruct((B, 128), jnp.float32),
        grid=(B // TB,),
        in_specs=[pl.BlockSpec((TB, 9 * 512), lambda i: (i, 0)),
                  pl.BlockSpec((9 * 512, 128), lambda i: (0, 0)),
                  pl.BlockSpec((1, 128), lambda i: (0, 0)),
                  pl.BlockSpec((128, 128), lambda i: (0, 0)),
                  pl.BlockSpec((1, 128), lambda i: (0, 0))],
        out_specs=pl.BlockSpec((TB, 128), lambda i: (i, 0)),
        compiler_params=pltpu.CompilerParams(
            dimension_semantics=("parallel",), vmem_limit_bytes=_VMEM),
    )(feats, w1r, fc1_b, fc2_w, fc2_b)
    return probs[:, :5]
